# Initial kernel scaffold; baseline (speedup 1.0000x reference)
import jax
import jax.numpy as jnp
from jax.experimental import pallas as pl


def kernel(x, x_mark_enc, x_dec, x_mark_dec, Wg_z, bg_z, Wl_z, bl_z, Wg_r, bg_r, Wl_r, bl_r, Wg_h, bg_h, Wl_h, bl_h, W_dec1, b_dec1, W_dec2, b_dec2):
    raise NotImplementedError("write your pallas kernel here")


# speedup vs baseline: 2.2681x; 2.2681x over previous
import functools

import jax
import jax.numpy as jnp
from jax.experimental import pallas as pl
from jax.experimental.pallas import tpu as pltpu

B = 64
D_IN = 512
D_MODEL = 1024
D_FF = 2048
PRED_LEN = 64
C_OUT = 128
N_OUT = PRED_LEN * C_OUT
BLOCK_N = 1024


def _mlp_kernel(x_ref, wgz_ref, bgz_ref, wgh_ref, bgh_ref,
                wlz_ref, blz_ref, wlh_ref, blh_ref,
                wd1_ref, bd1_ref, wd2_ref, bd2_ref,
                out_ref, y1_scr):
    j = pl.program_id(0)

    @pl.when(j == 0)
    def _stage12():
        x = x_ref[...]
        gz = jnp.dot(x, wgz_ref[...], preferred_element_type=jnp.float32) + bgz_ref[...]
        gh = jnp.dot(x, wgh_ref[...], preferred_element_type=jnp.float32) + bgh_ref[...]
        z = jax.nn.sigmoid(
            jnp.dot(gz, wlz_ref[...], preferred_element_type=jnp.float32) + blz_ref[...])
        ht = jnp.tanh(
            jnp.dot(gh, wlh_ref[...], preferred_element_type=jnp.float32) + blh_ref[...])
        h = jax.nn.relu((1.0 - z) * ht)
        y1_scr[...] = jax.nn.relu(
            jnp.dot(h, wd1_ref[...], preferred_element_type=jnp.float32) + bd1_ref[...])

    out_ref[...] = (
        jnp.dot(y1_scr[...], wd2_ref[...], preferred_element_type=jnp.float32)
        + bd2_ref[...])


@functools.partial(jax.jit, static_argnames=())
def _run(x2, Wg_z, bg_z, Wg_h, bg_h, Wl_z, bl_z, Wl_h, bl_h,
         W_dec1, b_dec1, W_dec2, b_dec2):
    n_blocks = N_OUT // BLOCK_N
    fixed = lambda j: (0, 0)
    y = pl.pallas_call(
        _mlp_kernel,
        grid=(n_blocks,),
        in_specs=[
            pl.BlockSpec((B, D_IN), fixed),
            pl.BlockSpec((D_IN, D_MODEL), fixed),
            pl.BlockSpec((1, D_MODEL), fixed),
            pl.BlockSpec((D_IN, D_MODEL), fixed),
            pl.BlockSpec((1, D_MODEL), fixed),
            pl.BlockSpec((D_MODEL, D_MODEL), fixed),
            pl.BlockSpec((1, D_MODEL), fixed),
            pl.BlockSpec((D_MODEL, D_MODEL), fixed),
            pl.BlockSpec((1, D_MODEL), fixed),
            pl.BlockSpec((D_MODEL, D_FF), fixed),
            pl.BlockSpec((1, D_FF), fixed),
            pl.BlockSpec((D_FF, BLOCK_N), lambda j: (0, j)),
            pl.BlockSpec((1, BLOCK_N), lambda j: (0, j)),
        ],
        out_specs=pl.BlockSpec((B, BLOCK_N), lambda j: (0, j)),
        out_shape=jax.ShapeDtypeStruct((B, N_OUT), jnp.float32),
        scratch_shapes=[pltpu.VMEM((B, D_FF), jnp.float32)],
    )(x2, Wg_z, bg_z.reshape(1, D_MODEL), Wg_h, bg_h.reshape(1, D_MODEL),
      Wl_z, bl_z.reshape(1, D_MODEL), Wl_h, bl_h.reshape(1, D_MODEL),
      W_dec1, b_dec1.reshape(1, D_FF), W_dec2, b_dec2.reshape(1, N_OUT))
    return y.reshape(B, PRED_LEN, C_OUT)


def kernel(x, x_mark_enc, x_dec, x_mark_dec, Wg_z, bg_z, Wl_z, bl_z,
           Wg_r, bg_r, Wl_r, bl_r, Wg_h, bg_h, Wl_h, bl_h,
           W_dec1, b_dec1, W_dec2, b_dec2):
    x2 = x.reshape(B, D_IN)
    return _run(x2, Wg_z, bg_z, Wg_h, bg_h, Wl_z, bl_z, Wl_h, bl_h,
                W_dec1, b_dec1, W_dec2, b_dec2)
